# baseline (speedup 1.0000x reference)
import jax
import jax.numpy as jnp
from jax.experimental import pallas as pl
from jax.experimental.pallas import tpu as pltpu

N_TASKS_K = 8
LATENT_K = 1024
OUT_K = 128
BLK = 512


def _body(task_ref, x_ref, w_ref, bias_ref, out_ref):
    y = jnp.dot(x_ref[...], w_ref[...], preferred_element_type=jnp.float32)
    y = y + bias_ref[...]
    tb = task_ref[0, 0, :]
    for t in range(N_TASKS_K):
        m = (tb == t).astype(jnp.float32)[:, None]
        out_ref[t, :, :] = y[:, t * OUT_K:(t + 1) * OUT_K] * m


def kernel(output_latents, output_task_index, W, b):
    Bsz, T, D = output_latents.shape
    N = Bsz * T
    x = output_latents.reshape(N, D)
    task = output_task_index.reshape(N).astype(jnp.int32)
    nblk = N // BLK
    task3 = task.reshape(nblk, 1, BLK)
    w_all = jnp.transpose(W, (2, 0, 1)).reshape(D, N_TASKS_K * OUT_K)
    bias_row = b.reshape(1, N_TASKS_K * OUT_K)

    out = pl.pallas_call(
        _body,
        grid=(nblk,),
        in_specs=[
            pl.BlockSpec((1, 1, BLK), lambda i: (i, 0, 0)),
            pl.BlockSpec((BLK, D), lambda i: (i, 0)),
            pl.BlockSpec((D, N_TASKS_K * OUT_K), lambda i: (0, 0)),
            pl.BlockSpec((1, N_TASKS_K * OUT_K), lambda i: (0, 0)),
        ],
        out_specs=pl.BlockSpec((N_TASKS_K, BLK, OUT_K), lambda i: (0, i, 0)),
        out_shape=jax.ShapeDtypeStruct((N_TASKS_K, N, OUT_K), jnp.float32),
    )(task3, x, w_all, bias_row)
    return out.reshape(N_TASKS_K, Bsz, T, OUT_K)


# speedup vs baseline: 1.1324x; 1.1324x over previous
import jax
import jax.numpy as jnp
from jax.experimental import pallas as pl
from jax.experimental.pallas import tpu as pltpu

N_TASKS_K = 8
LATENT_K = 1024
OUT_K = 128
BLK = 2048


def _body(task_ref, x_ref, w_ref, bias_ref, out_ref):
    y = jnp.dot(x_ref[...], w_ref[...], preferred_element_type=jnp.float32)
    y = y + bias_ref[...]
    tb = task_ref[0, 0, :]
    for t in range(N_TASKS_K):
        m = (tb == t).astype(jnp.float32)[:, None]
        out_ref[t, :, :] = y[:, t * OUT_K:(t + 1) * OUT_K] * m


def kernel(output_latents, output_task_index, W, b):
    Bsz, T, D = output_latents.shape
    N = Bsz * T
    x = output_latents.reshape(N, D)
    task = output_task_index.reshape(N).astype(jnp.int32)
    nblk = N // BLK
    task3 = task.reshape(nblk, 1, BLK)
    w_all = jnp.transpose(W, (2, 0, 1)).reshape(D, N_TASKS_K * OUT_K)
    bias_row = b.reshape(1, N_TASKS_K * OUT_K)

    out = pl.pallas_call(
        _body,
        grid=(nblk,),
        in_specs=[
            pl.BlockSpec((1, 1, BLK), lambda i: (i, 0, 0)),
            pl.BlockSpec((BLK, D), lambda i: (i, 0)),
            pl.BlockSpec((D, N_TASKS_K * OUT_K), lambda i: (0, 0)),
            pl.BlockSpec((1, N_TASKS_K * OUT_K), lambda i: (0, 0)),
        ],
        out_specs=pl.BlockSpec((N_TASKS_K, BLK, OUT_K), lambda i: (0, i, 0)),
        out_shape=jax.ShapeDtypeStruct((N_TASKS_K, N, OUT_K), jnp.float32),
    )(task3, x, w_all, bias_row)
    return out.reshape(N_TASKS_K, Bsz, T, OUT_K)


# speedup vs baseline: 1.1645x; 1.0284x over previous
import jax
import jax.numpy as jnp
from jax.experimental import pallas as pl
from jax.experimental.pallas import tpu as pltpu

N_TASKS_K = 8
LATENT_K = 1024
OUT_K = 128
BLK = 2048


def _body(task_ref, x_ref, w_ref, bias_ref, out_ref):
    y = jnp.dot(x_ref[...].astype(jnp.bfloat16), w_ref[...],
                preferred_element_type=jnp.float32)
    y = y + bias_ref[...]
    tb = task_ref[0, 0, :]
    for t in range(N_TASKS_K):
        m = (tb == t).astype(jnp.float32)[:, None]
        out_ref[t, :, :] = y[:, t * OUT_K:(t + 1) * OUT_K] * m


def kernel(output_latents, output_task_index, W, b):
    Bsz, T, D = output_latents.shape
    N = Bsz * T
    x = output_latents.reshape(N, D)
    task = output_task_index.reshape(N).astype(jnp.int32)
    nblk = N // BLK
    task3 = task.reshape(nblk, 1, BLK)
    w_all = jnp.transpose(W, (2, 0, 1)).reshape(D, N_TASKS_K * OUT_K)
    w_all = w_all.astype(jnp.bfloat16)
    bias_row = b.reshape(1, N_TASKS_K * OUT_K)

    out = pl.pallas_call(
        _body,
        grid=(nblk,),
        in_specs=[
            pl.BlockSpec((1, 1, BLK), lambda i: (i, 0, 0)),
            pl.BlockSpec((BLK, D), lambda i: (i, 0)),
            pl.BlockSpec((D, N_TASKS_K * OUT_K), lambda i: (0, 0)),
            pl.BlockSpec((1, N_TASKS_K * OUT_K), lambda i: (0, 0)),
        ],
        out_specs=pl.BlockSpec((N_TASKS_K, BLK, OUT_K), lambda i: (0, i, 0)),
        out_shape=jax.ShapeDtypeStruct((N_TASKS_K, N, OUT_K), jnp.float32),
    )(task3, x, w_all, bias_row)
    return out.reshape(N_TASKS_K, Bsz, T, OUT_K)
